# baseline (speedup 1.0000x reference)
import jax
import jax.numpy as jnp
from jax import lax
from jax.experimental import pallas as pl
from jax.experimental.pallas import tpu as pltpu

S_ROI = 14
K_PER_LEVEL = 128
NUM_CLASSES = 80
MAX_OUT = 100
IOU_THR = 0.45
SCORE_THR = 0.5
DW_RATE = (4.0, 8.0, 16.0, 32.0, 64.0)
LEVEL_HW = [(128, 128), (64, 64), (32, 32), (16, 16), (8, 8)]
ANCHOR_HW = ((8.0, 8.0), (4.0, 16.0), (16.0, 4.0))

NBOX = 5 * K_PER_LEVEL


def _conv(x, w, b):
    return lax.conv_general_dilated(x, w, (1, 1), 'SAME',
                                    dimension_numbers=('NHWC', 'HWIO', 'NHWC')) + b


def _make_anchors(H, W):
    cy = jnp.arange(H, dtype=jnp.float32) + 0.5
    cx = jnp.arange(W, dtype=jnp.float32) + 0.5
    hw = jnp.asarray(ANCHOR_HW, jnp.float32)
    cy = jnp.broadcast_to(cy[:, None, None], (H, W, 3))
    cx = jnp.broadcast_to(cx[None, :, None], (H, W, 3))
    h, w = hw[:, 0], hw[:, 1]
    return jnp.stack([cy - h / 2, cx - w / 2, cy + h / 2, cx + w / 2], -1).reshape(-1, 4)


def _bbox_decode(boxes, deltas, norm):
    ny, nx, nh, nw = norm
    cy = (boxes[:, 0] + boxes[:, 2]) * 0.5
    cx = (boxes[:, 1] + boxes[:, 3]) * 0.5
    h = boxes[:, 2] - boxes[:, 0]
    w = boxes[:, 3] - boxes[:, 1]
    dy = deltas[:, 0] / ny
    dx = deltas[:, 1] / nx
    dh = jnp.clip(deltas[:, 2] / nh, -4.0, 4.0)
    dw = jnp.clip(deltas[:, 3] / nw, -4.0, 4.0)
    cy = cy + dy * h
    cx = cx + dx * w
    h = h * jnp.exp(dh)
    w = w * jnp.exp(dw)
    return jnp.stack([cy - h * 0.5, cx - w * 0.5, cy + h * 0.5, cx + w * 0.5], -1)


def _roi_align(feat, boxes):
    H, W, _ = feat.shape
    g = (jnp.arange(S_ROI, dtype=jnp.float32) + 0.5) / S_ROI
    ys = boxes[:, 0:1] + g[None, :] * (boxes[:, 2:3] - boxes[:, 0:1]) - 0.5
    xs = boxes[:, 1:2] + g[None, :] * (boxes[:, 3:4] - boxes[:, 1:2]) - 0.5
    y0 = jnp.floor(ys)
    x0 = jnp.floor(xs)
    wy = (ys - y0)[:, :, None, None]
    wx = (xs - x0)[:, None, :, None]
    y0i = jnp.clip(y0, 0, H - 1).astype(jnp.int32)
    y1i = jnp.clip(y0 + 1, 0, H - 1).astype(jnp.int32)
    x0i = jnp.clip(x0, 0, W - 1).astype(jnp.int32)
    x1i = jnp.clip(x0 + 1, 0, W - 1).astype(jnp.int32)
    gat = lambda yi, xi: feat[yi[:, :, None], xi[:, None, :]]
    return (gat(y0i, x0i) * (1 - wy) * (1 - wx) + gat(y0i, x1i) * (1 - wy) * wx
            + gat(y1i, x0i) * wy * (1 - wx) + gat(y1i, x1i) * wy * wx)


def _nms_kernel(sc_ref, box_ref, out_ref):
    y1 = box_ref[0]
    x1 = box_ref[1]
    y2 = box_ref[2]
    x2 = box_ref[3]
    areas = (y2 - y1) * (x2 - x1)
    rows = lax.broadcasted_iota(jnp.int32, (5, 128), 0)
    cols = lax.broadcasted_iota(jnp.int32, (5, 128), 1)
    pid = rows * 128 + cols
    lane3 = lax.broadcasted_iota(jnp.int32, (1, 1, 128), 2)

    def body(k, s):
        m = jnp.max(s)
        v = m > SCORE_THR
        idx = jnp.min(jnp.where(s == m, pid, jnp.int32(1 << 30)))
        sel = pid == idx
        by1 = jnp.sum(jnp.where(sel, y1, 0.0))
        bx1 = jnp.sum(jnp.where(sel, x1, 0.0))
        by2 = jnp.sum(jnp.where(sel, y2, 0.0))
        bx2 = jnp.sum(jnp.where(sel, x2, 0.0))
        yy1 = jnp.maximum(by1, y1)
        xx1 = jnp.maximum(bx1, x1)
        yy2 = jnp.minimum(by2, y2)
        xx2 = jnp.minimum(bx2, x2)
        inter = jnp.clip(yy2 - yy1, 0.0) * jnp.clip(xx2 - xx1, 0.0)
        a = (by2 - by1) * (bx2 - bx1)
        iou = inter / (a + areas - inter + 1e-9)
        row = jnp.where(lane3 == 0, jnp.where(v, m, 0.0), 0.0)
        row = jnp.where(lane3 == 1, jnp.where(v, by1, 0.0), row)
        row = jnp.where(lane3 == 2, jnp.where(v, bx1, 0.0), row)
        row = jnp.where(lane3 == 3, jnp.where(v, by2, 0.0), row)
        row = jnp.where(lane3 == 4, jnp.where(v, bx2, 0.0), row)
        row = jnp.where(lane3 == 5, jnp.where(v, 1.0, 0.0), row)
        out_ref[pl.ds(k, 1), :, :] = row
        s_sup = jnp.where(iou > IOU_THR, -1.0, s)
        s = jnp.where(v, s_sup, s)
        s = jnp.where(sel, -1.0, s)
        return s

    lax.fori_loop(0, MAX_OUT, body, sc_ref[0])


def _nms_all_classes(sc, pred):
    sc_t = sc.T.reshape(NUM_CLASSES, 5, 128)
    box_t = pred.T.reshape(4, 5, 128)
    out = pl.pallas_call(
        _nms_kernel,
        grid=(NUM_CLASSES,),
        in_specs=[
            pl.BlockSpec((1, 5, 128), lambda c: (c, 0, 0)),
            pl.BlockSpec((4, 5, 128), lambda c: (0, 0, 0)),
        ],
        out_specs=pl.BlockSpec((MAX_OUT, 1, 128), lambda c: (c, 0, 0)),
        out_shape=jax.ShapeDtypeStruct((NUM_CLASSES * MAX_OUT, 1, 128), jnp.float32),
        compiler_params=pltpu.CompilerParams(
            dimension_semantics=('parallel',)),
    )(sc_t, box_t)
    out = out.reshape(NUM_CLASSES, MAX_OUT, 128)
    sel_sc = out[:, :, 0]
    sel_bx = out[:, :, 1:5]
    sel_vd = out[:, :, 5] > 0.5
    return sel_sc, sel_bx, sel_vd


def kernel(f0, f1, f2, f3, f4, rpn_w1, rpn_b1, rpn_w2, rpn_b2, rpn_wo, rpn_bo,
           rcnn_w1, rcnn_b1, rcnn_w2, rcnn_b2, rcnn_w3, rcnn_b3,
           cls_w, cls_b, reg_w, reg_b):
    feats = [f0, f1, f2, f3, f4]
    rois, props, valids = [], [], []
    for l, f in enumerate(feats):
        H, W, _ = f.shape
        x = f[None]
        c = jax.nn.relu(_conv(x, rpn_w1[l], rpn_b1[l]))
        c = jax.nn.relu(_conv(c, rpn_w2[l], rpn_b2[l]))
        o = _conv(c, rpn_wo[l], rpn_bo[l])[0].reshape(-1, 6)
        prob = jax.nn.softmax(o[:, :2], axis=-1)[:, 1]
        reg = o[:, 2:]
        anchors = _make_anchors(H, W)
        topv, topi = lax.top_k(prob, K_PER_LEVEL)
        prop = _bbox_decode(anchors[topi], reg[topi], (10.0, 10.0, 5.0, 5.0))
        rois.append(_roi_align(f, prop))
        props.append(prop * DW_RATE[l])
        valids.append(topv > 0.5)
    roi = jnp.concatenate(rois, 0)
    prop = jnp.concatenate(props, 0)
    valid = jnp.concatenate(valids, 0)
    c = jax.nn.relu(_conv(roi, rcnn_w1, rcnn_b1))
    c = jax.nn.relu(_conv(c, rcnn_w2, rcnn_b2))
    c = jax.nn.relu(_conv(c, rcnn_w3, rcnn_b3))
    flat = c.reshape(c.shape[0], -1)
    cla = flat @ cls_w + cls_b
    reg2 = flat @ reg_w + reg_b
    pred = _bbox_decode(prop, reg2, (1.0, 1.0, 1.0, 1.0))
    sc = jax.nn.sigmoid(cla)
    sc = jnp.where(valid[:, None], sc, 0.0)
    sc = jnp.where(sc > SCORE_THR, sc, -1.0)
    p = jnp.isfinite(sc[0, 0])

    def _true_branch(ops):
        return _nms_all_classes(*ops)

    def _false_branch(ops):
        z = jnp.zeros((NUM_CLASSES, MAX_OUT), jnp.float32)
        return z, jnp.zeros((NUM_CLASSES, MAX_OUT, 4), jnp.float32), z > 1.0

    sel_sc, sel_bx, sel_vd = lax.cond(p, _true_branch, _false_branch, (sc, pred))
    class_id = jnp.where(sel_vd, jnp.arange(NUM_CLASSES, dtype=jnp.int32)[:, None], -1).reshape(-1)
    return class_id, sel_sc.reshape(-1), sel_bx.reshape(-1, 4)
